# baseline (speedup 1.0000x reference)
import jax
import jax.numpy as jnp
from jax import lax
from jax.experimental import pallas as pl
from jax.experimental.pallas import tpu as pltpu
from jax.experimental.pallas import tpu_sc as plsc

_NC = 2
_NS = 16
_NW = _NC * _NS

_B, _L = 4096, 200
_N = _B * _L
_V = 21
_D = 21
_DP = 24
_IW = 128
_PER_W = _N // _NW
_TILES_W = _PER_W // _IW
_G = 20
_NGRP = _TILES_W // _G
_R = 256
_PKW = _PER_W // 2
_PKR = _PKW // _IW
_HALF = _PER_W // 2


def _body(idx_hbm, table_hbm, out_hbm, idx_pk, idx_all, sem_g, sem_o,
          *row_bufs):
    wid = lax.axis_index("s") * _NC + lax.axis_index("c")
    pltpu.sync_copy(idx_hbm.at[wid], idx_pk)

    def unpack_row(r, carry):
        for g in range(8):
            pk = idx_pk[r, pl.ds(g * 16, 16)]
            lo = lax.bitwise_and(pk, jnp.int32(0xFFFF))
            hi = lax.shift_right_logical(pk, jnp.int32(16))
            f = r * 128 + g * 16
            f2 = _HALF + f
            idx_all[f // 1024, (f // 128) % 8, pl.ds(f % 128, 16)] = lo
            idx_all[f2 // 1024, (f2 // 128) % 8, pl.ds(f2 % 128, 16)] = hi
        return carry

    lax.fori_loop(0, _PKR, unpack_row, 0)

    def drain_writes():
        for j in range(_G):
            pltpu.make_async_copy(row_bufs[j], out_hbm.at[pl.ds(0, _IW)],
                                  sem_o).wait()

    def step(i, carry):
        @pl.when(i > 0)
        def _():
            drain_writes()

        t0 = i * _G
        gathers = [
            pltpu.async_copy(
                table_hbm.at[idx_all.at[(t0 + j) // 8, (t0 + j) % 8]],
                row_bufs[j], sem_g)
            for j in range(_G)
        ]
        for g in gathers:
            g.wait()
        base = wid * _PER_W + t0 * _IW
        for j in range(_G):
            pltpu.async_copy(row_bufs[j],
                             out_hbm.at[pl.ds(base + j * _IW, _IW)], sem_o)
        return carry

    lax.fori_loop(0, _NGRP, step, 0)
    drain_writes()


def kernel(sequence_indices, table):
    rep_off = _V * (jnp.arange(_N, dtype=jnp.int32) % _R)
    idxo = (sequence_indices.reshape(_N) + rep_off).reshape(_NW, 2, _HALF)
    packed = (idxo[:, 0, :] | (idxo[:, 1, :] << 16)).reshape(
        _NW, _PKR, _IW)
    table_padded = jnp.tile(jnp.pad(table, ((0, 0), (0, _DP - _D))),
                            (_R, 1))
    mesh = plsc.VectorSubcoreMesh(
        core_axis_name="c", subcore_axis_name="s",
        num_cores=_NC, num_subcores=_NS,
    )
    k = pl.kernel(
        _body,
        out_type=jax.ShapeDtypeStruct((_N, _DP), jnp.float32),
        mesh=mesh,
        scratch_types=[
            pltpu.VMEM((_PKR, _IW), jnp.int32),
            pltpu.VMEM((_TILES_W // 8, 8, _IW), jnp.int32),
            pltpu.SemaphoreType.DMA,
            pltpu.SemaphoreType.DMA,
        ] + [pltpu.VMEM((_IW, _DP), jnp.float32) for _ in range(_G)],
        compiler_params=pltpu.CompilerParams(use_tc_tiling_on_sc=False),
    )
    out = k(packed, table_padded)
    return out[:, :_D].reshape(_B, _L, _D)


# speedup vs baseline: 1.0091x; 1.0091x over previous
import jax
import jax.numpy as jnp
from jax import lax
from jax.experimental import pallas as pl
from jax.experimental.pallas import tpu as pltpu
from jax.experimental.pallas import tpu_sc as plsc

_NC = 2
_NS = 16
_NW = _NC * _NS

_B, _L = 4096, 200
_N = _B * _L
_V = 21
_D = 21
_DP = 24
_IW = 128
_PER_W = _N // _NW
_TILES_W = _PER_W // _IW
_G = 20
_NGRP = _TILES_W // _G
_R = 256


def _body(idx_hbm, table_hbm, out_hbm, idx_all, sem_g, sem_o, *row_bufs):
    wid = lax.axis_index("s") * _NC + lax.axis_index("c")
    pltpu.sync_copy(idx_hbm.at[pl.ds(wid * (_TILES_W // 8), _TILES_W // 8)],
                    idx_all)

    def drain_writes():
        for j in range(_G):
            pltpu.make_async_copy(row_bufs[j], out_hbm.at[pl.ds(0, _IW)],
                                  sem_o).wait()

    def step(i, carry):
        @pl.when(i > 0)
        def _():
            drain_writes()

        t0 = i * _G
        gathers = [
            pltpu.async_copy(
                table_hbm.at[idx_all.at[(t0 + j) // 8, (t0 + j) % 8]],
                row_bufs[j], sem_g)
            for j in range(_G)
        ]
        for g in gathers:
            g.wait()
        base = wid * _PER_W + t0 * _IW
        for j in range(_G):
            pltpu.async_copy(row_bufs[j],
                             out_hbm.at[pl.ds(base + j * _IW, _IW)], sem_o)
        return carry

    lax.fori_loop(0, _NGRP, step, 0)
    drain_writes()


def kernel(sequence_indices, table):
    rep_off = _V * (jnp.arange(_N, dtype=jnp.int32) % _R)
    idx_rows = (sequence_indices.reshape(_N) + rep_off).reshape(
        _N // 1024, 8, _IW)
    table_padded = jnp.tile(jnp.pad(table, ((0, 0), (0, _DP - _D))),
                            (_R, 1))
    mesh = plsc.VectorSubcoreMesh(
        core_axis_name="c", subcore_axis_name="s",
        num_cores=_NC, num_subcores=_NS,
    )
    k = pl.kernel(
        _body,
        out_type=jax.ShapeDtypeStruct((_N, _DP), jnp.float32),
        mesh=mesh,
        scratch_types=[
            pltpu.VMEM((_TILES_W // 8, 8, _IW), jnp.int32),
            pltpu.SemaphoreType.DMA,
            pltpu.SemaphoreType.DMA,
        ] + [pltpu.VMEM((_IW, _DP), jnp.float32) for _ in range(_G)],
        compiler_params=pltpu.CompilerParams(use_tc_tiling_on_sc=False),
    )
    out = k(idx_rows, table_padded)
    return out[:, :_D].reshape(_B, _L, _D)
